# baseline (speedup 1.0000x reference)
import jax, jax.numpy as jnp
from jax import lax
from jax.experimental import pallas as pl
from jax.experimental.pallas import tpu as pltpu
from jax.experimental.pallas import tpu_sc as plsc

B, D = 2048, 16384
BPW = 64
NSEM = 8
NC, NS = 2, 16


def _body(t4, idx_hbm, out4, idx_v, sems):
    wid = lax.axis_index("s") * NC + lax.axis_index("c")
    base = wid * BPW

    pltpu.sync_copy(idx_hbm.at[pl.ds(base, BPW)], idx_v)
    rows = []
    for blk in range(BPW // 16):
        v = idx_v[pl.ds(blk * 16, 16)]
        rows.extend(v[j] for j in range(16))

    def start_copy(g):
        r = rows[g]
        pltpu.async_copy(
            t4.at[pl.ds(r // 8, 1), :, pl.ds(r % 8, 1), :],
            out4.at[pl.ds(wid * 8 + g // 8, 1), :, pl.ds(g % 8, 1), :],
            sems.at[g % NSEM],
        )

    def wait_copy(g):
        pltpu.make_async_copy(
            t4.at[pl.ds(0, 1), :, pl.ds(0, 1), :],
            out4.at[pl.ds(0, 1), :, pl.ds(0, 1), :],
            sems.at[g % NSEM],
        ).wait()

    for g in range(NSEM):
        start_copy(g)
    for g in range(BPW):
        wait_copy(g)
        if g + NSEM < BPW:
            start_copy(g + NSEM)


def kernel(prefix, table):
    idx = prefix.reshape(B)
    t4 = table.reshape(128, 8, 128, 128).transpose(0, 2, 1, 3)
    mesh = plsc.VectorSubcoreMesh(core_axis_name="c", subcore_axis_name="s",
                                  num_cores=NC, num_subcores=NS)
    f = pl.kernel(
        _body,
        out_type=jax.ShapeDtypeStruct((B // 8, 128, 8, 128), jnp.float32),
        mesh=mesh,
        scratch_types=[pltpu.VMEM((BPW,), jnp.int32),
                       pltpu.SemaphoreType.DMA((NSEM,))],
    )
    out4 = f(t4, idx)
    return out4.transpose(0, 2, 1, 3).reshape(16, 128, D)


# speedup vs baseline: 36.2080x; 36.2080x over previous
import jax
import jax.numpy as jnp
from jax import lax
from jax.experimental import pallas as pl
from jax.experimental.pallas import tpu as pltpu
from jax.experimental.pallas import tpu_sc as plsc

B, D = 2048, 16384
BPW = 64
NBUF = 7
LA = 5
NC, NS = 2, 16


def _body(t4, idx_hbm, out4, idx_v, *rest):
    bufs = rest[:NBUF]
    in_sems, out_sems = rest[NBUF], rest[NBUF + 1]
    wid = lax.axis_index("s") * NC + lax.axis_index("c")
    base = wid * BPW

    pltpu.sync_copy(idx_hbm.at[pl.ds(base, BPW)], idx_v)
    rows = []
    for blk in range(BPW // 16):
        v = idx_v[pl.ds(blk * 16, 16)]
        rows.extend(v[j] for j in range(16))

    def start_gather(g, b):
        r = rows[g]
        pltpu.async_copy(
            t4.at[pl.ds(r // 8, 1), :, pl.ds(r % 8, 1), :], bufs[b],
            in_sems.at[b],
        )

    def wait_gather(b):
        pltpu.make_async_copy(
            t4.at[pl.ds(0, 1), :, pl.ds(0, 1), :], bufs[b], in_sems.at[b]
        ).wait()

    def start_out(g, b):
        pltpu.async_copy(
            bufs[b],
            out4.at[pl.ds(wid * 8 + g // 8, 1), :, pl.ds(g % 8, 1), :],
            out_sems.at[b],
        )

    def wait_out(b):
        pltpu.make_async_copy(
            bufs[b], t4.at[pl.ds(0, 1), :, pl.ds(0, 1), :], out_sems.at[b]
        ).wait()

    for h in range(LA):
        start_gather(h, h % NBUF)
    for g in range(BPW):
        b = g % NBUF
        wait_gather(b)
        start_out(g, b)
        h = g + LA
        if h < BPW:
            bh = h % NBUF
            if h >= NBUF:
                wait_out(bh)
            start_gather(h, bh)
    for g in range(BPW - NBUF, BPW):
        wait_out(g % NBUF)


def kernel(prefix, table):
    idx = prefix.reshape(B)
    t4 = table.reshape(128, 8, 128, 128).transpose(0, 2, 1, 3)
    mesh = plsc.VectorSubcoreMesh(core_axis_name="c", subcore_axis_name="s",
                                  num_cores=NC, num_subcores=NS)
    f = pl.kernel(
        _body,
        out_type=jax.ShapeDtypeStruct((B // 8, 128, 8, 128), jnp.float32),
        mesh=mesh,
        scratch_types=(
            [pltpu.VMEM((BPW,), jnp.int32)]
            + [pltpu.VMEM((1, 128, 1, 128), jnp.float32) for _ in range(NBUF)]
            + [pltpu.SemaphoreType.DMA((NBUF,)),
               pltpu.SemaphoreType.DMA((NBUF,))]
        ),
    )
    out4 = f(t4, idx)
    return out4.transpose(0, 2, 1, 3).reshape(16, 128, D)
